# baseline (speedup 1.0000x reference)
import jax
import jax.numpy as jnp
from jax import lax
from jax.experimental import pallas as pl
from jax.experimental.pallas import tpu as pltpu

_NEG_SLOPE = 0.2


def _phase_a(x, w, bt, n, c):
    hat = lax.dot_general(
        w, x, dimension_numbers=(((0,), (1,)), ((), ())),
        preferred_element_type=jnp.float32,
    ) + bt
    h3 = hat[:c, :]
    f3 = jnp.minimum(jnp.min(h3, axis=-1, keepdims=True), 0.0)
    g3 = jnp.maximum(jnp.max(h3, axis=-1, keepdims=True), 0.0)
    lg = hat[c:c + 1, :]
    lo = jnp.maximum(lg, _NEG_SLOPE * lg)
    t = jnp.exp(lo - jnp.max(lo))
    tht = h3 * t

    ones_row = jnp.ones((1, n), jnp.float32)
    pad_rt = jnp.zeros((8 - (c + 2), n), jnp.float32)
    pad_uv = jnp.zeros((8 - 2 * c, n), jnp.float32)
    rt8 = jnp.concatenate([tht, t, ones_row, pad_rt], axis=0)
    uv8 = jnp.concatenate([tht - f3, tht - g3, pad_uv], axis=0)
    fg8 = jnp.concatenate([f3, g3, jnp.zeros((8 - 2 * c, 1), jnp.float32)],
                          axis=0)
    return rt8.astype(jnp.bfloat16), uv8.astype(jnp.bfloat16), fg8


def _bulk(x, adj, rt8, uv8, fg8, skip_sel, c):
    n = adj.shape[-1]
    adjb = adj.astype(jnp.bfloat16)

    psT = lax.dot_general(
        rt8, adjb, dimension_numbers=(((1,), (1,)), ((), ())),
        preferred_element_type=jnp.float32,
    )
    invr = 1.0 / psT[c:c + 1, :]
    rdeg = 1.0 / jnp.maximum(psT[c + 1:c + 2, :], 1.0)
    sT = psT[:c, :] * invr
    mT = sT * rdeg
    ms6 = jnp.concatenate([mT, sT], axis=0).T

    id2 = jnp.concatenate([invr, psT[c + 1:c + 2, :]], axis=0).T
    inv_col = id2[:, 0:1]
    deg_col = id2[:, 1:2]

    ub = uv8[:c, :]
    vb = uv8[c:2 * c, :]
    blk = 64
    parts = []
    for r0 in range(0, n, blk):
        ab = adjb[r0:r0 + blk, :]
        mms = []
        for ch in range(c):
            mms.append(jnp.max(ab * ub[ch:ch + 1, :], axis=-1, keepdims=True))
        for ch in range(c):
            mms.append(jnp.min(ab * vb[ch:ch + 1, :], axis=-1, keepdims=True))
        parts.append(jnp.concatenate(mms, axis=-1))
    fg6 = fg8[:2 * c, :].T
    mm6 = (jnp.concatenate(parts, axis=0).astype(jnp.float32) + fg6) * inv_col

    agg = jnp.concatenate(
        [ms6[:, :c], mm6[:, :c], ms6[:, c:], mm6[:, c:]], axis=-1)
    agg = jnp.where(deg_col > 0.0, agg, jnp.zeros_like(agg))

    skip = jnp.dot(x, skip_sel, preferred_element_type=jnp.float32)
    return agg + skip


def _gconv_kernel(x_ref, x2_ref, adj_ref, w_ref, b_ref, out_ref,
                  rt_s, uv_s, fg_s):
    n = adj_ref.shape[-1]
    c = out_ref.shape[-1] // 4
    w = w_ref[...]
    bt = b_ref[...].T
    rows = lax.broadcasted_iota(jnp.int32, (c, 4 * c), 0)
    cols = lax.broadcasted_iota(jnp.int32, (c, 4 * c), 1)
    skip_sel = (cols % c == rows).astype(jnp.float32)

    i = pl.program_id(0)
    slot = lax.rem(i, 2)
    nslot = lax.rem(i + 1, 2)

    @pl.when(i == 0)
    def _():
        r0, u0, f0 = _phase_a(x_ref[...], w, bt, n, c)
        rt_s[0] = r0
        uv_s[0] = u0
        fg_s[0, :, 0:1] = f0

    out_ref[...] = _bulk(x_ref[...], adj_ref[...], rt_s[slot], uv_s[slot],
                         fg_s[slot, :, 0:1], skip_sel, c)

    r1, u1, f1 = _phase_a(x2_ref[...], w, bt, n, c)
    rt_s[nslot] = r1
    uv_s[nslot] = u1
    fg_s[nslot, :, 0:1] = f1


@jax.jit
def _forward(x, adj, w_aug, b_aug):
    bsz, n, c_in = x.shape
    c = w_aug.shape[1] - 1
    return pl.pallas_call(
        _gconv_kernel,
        out_shape=jax.ShapeDtypeStruct((bsz, n, 4 * c), jnp.float32),
        grid=(bsz,),
        in_specs=[
            pl.BlockSpec((None, n, c_in), lambda i: (i, 0, 0)),
            pl.BlockSpec((None, n, c_in),
                         lambda i: (jnp.minimum(i + 1, bsz - 1), 0, 0)),
            pl.BlockSpec((None, n, n), lambda i: (i, 0, 0)),
            pl.BlockSpec((c_in, c + 1), lambda i: (0, 0)),
            pl.BlockSpec((1, c + 1), lambda i: (0, 0)),
        ],
        out_specs=pl.BlockSpec((None, n, 4 * c), lambda i: (i, 0, 0)),
        scratch_shapes=[
            pltpu.VMEM((2, 8, n), jnp.bfloat16),
            pltpu.VMEM((2, 8, n), jnp.bfloat16),
            pltpu.VMEM((2, 8, 128), jnp.float32),
        ],
        compiler_params=pltpu.CompilerParams(
            dimension_semantics=("arbitrary",)),
    )(x, x, adj, w_aug, b_aug)


def kernel(x, adj, w_aug, b_aug):
    return _forward(x, adj, w_aug, b_aug)


# speedup vs baseline: 1.0054x; 1.0054x over previous
import jax
import jax.numpy as jnp
from jax import lax
from jax.experimental import pallas as pl
from jax.experimental.pallas import tpu as pltpu

_NEG_SLOPE = 0.2


def _gconv_kernel(x_ref, adj_ref, w_ref, b_ref, out_ref):
    n = adj_ref.shape[-1]
    c = out_ref.shape[-1] // 4
    x = x_ref[...]
    adj = adj_ref[...]

    adjb = adj.astype(jnp.bfloat16)

    hat = lax.dot_general(
        w_ref[...], x, dimension_numbers=(((0,), (1,)), ((), ())),
        preferred_element_type=jnp.float32,
    ) + b_ref[...].T
    h3 = hat[:c, :]
    f3 = jnp.minimum(jnp.min(h3, axis=-1, keepdims=True), 0.0)
    g3 = jnp.maximum(jnp.max(h3, axis=-1, keepdims=True), 0.0)
    lg = hat[c:c + 1, :]
    lo = jnp.maximum(lg, _NEG_SLOPE * lg)
    t = jnp.exp(lo - jnp.max(lo))
    tht = h3 * t

    ones_row = jnp.ones((1, n), jnp.float32)
    rt = jnp.concatenate([tht, t, ones_row], axis=0).astype(jnp.bfloat16)
    psT = lax.dot_general(
        rt, adjb, dimension_numbers=(((1,), (1,)), ((), ())),
        preferred_element_type=jnp.float32,
    )
    invr = 1.0 / psT[c:c + 1, :]
    rdeg = 1.0 / jnp.maximum(psT[c + 1:c + 2, :], 1.0)
    sT = psT[:c, :] * invr
    mT = sT * rdeg
    ms6 = jnp.concatenate([mT, sT], axis=0).T

    id2 = jnp.concatenate([invr, psT[c + 1:c + 2, :]], axis=0).T
    inv_col = id2[:, 0:1]
    deg_col = id2[:, 1:2]

    ub = (tht - f3).astype(jnp.bfloat16)
    vb = (tht - g3).astype(jnp.bfloat16)
    blk = 64
    parts = []
    for r0 in range(0, n, blk):
        ab = adjb[r0:r0 + blk, :]
        mms = []
        for ch in range(c):
            mms.append(jnp.max(ab * ub[ch:ch + 1, :], axis=-1, keepdims=True))
        for ch in range(c):
            mms.append(jnp.min(ab * vb[ch:ch + 1, :], axis=-1, keepdims=True))
        parts.append(jnp.concatenate(mms, axis=-1))
    fg6 = jnp.concatenate([f3, g3], axis=0).T
    mm6 = (jnp.concatenate(parts, axis=0).astype(jnp.float32) + fg6) * inv_col

    agg = jnp.concatenate(
        [ms6[:, :c], mm6[:, :c], ms6[:, c:], mm6[:, c:]], axis=-1)
    agg = jnp.where(deg_col > 0.0, agg, jnp.zeros_like(agg))

    rows = lax.broadcasted_iota(jnp.int32, (c, 4 * c), 0)
    cols = lax.broadcasted_iota(jnp.int32, (c, 4 * c), 1)
    skip_sel = (cols % c == rows).astype(jnp.float32)
    skip = jnp.dot(x, skip_sel, preferred_element_type=jnp.float32)
    out_ref[...] = agg + skip


@jax.jit
def _forward(x, adj, w_aug, b_aug):
    bsz, n, c_in = x.shape
    c = w_aug.shape[1] - 1
    return pl.pallas_call(
        _gconv_kernel,
        out_shape=jax.ShapeDtypeStruct((bsz, n, 4 * c), jnp.float32),
        grid=(bsz,),
        in_specs=[
            pl.BlockSpec((None, n, c_in), lambda i: (i, 0, 0)),
            pl.BlockSpec((None, n, n), lambda i: (i, 0, 0)),
            pl.BlockSpec((c_in, c + 1), lambda i: (0, 0)),
            pl.BlockSpec((1, c + 1), lambda i: (0, 0)),
        ],
        out_specs=pl.BlockSpec((None, n, 4 * c), lambda i: (i, 0, 0)),
        compiler_params=pltpu.CompilerParams(dimension_semantics=("parallel",)),
    )(x, adj, w_aug, b_aug)


def kernel(x, adj, w_aug, b_aug):
    return _forward(x, adj, w_aug, b_aug)


# speedup vs baseline: 1.0085x; 1.0031x over previous
import jax
import jax.numpy as jnp
from jax import lax
from jax.experimental import pallas as pl
from jax.experimental.pallas import tpu as pltpu

_NEG_SLOPE = 0.2


def _gconv_kernel(x_ref, adj_ref, w_ref, b_ref, out_ref):
    n = adj_ref.shape[-1]
    c = out_ref.shape[-1] // 4
    x = x_ref[...]
    adj = adj_ref[...]

    hat = lax.dot_general(
        w_ref[...], x, dimension_numbers=(((0,), (1,)), ((), ())),
        preferred_element_type=jnp.float32,
    ) + b_ref[...].T
    h3 = hat[:c, :]
    f3 = jnp.minimum(jnp.min(h3, axis=-1, keepdims=True), 0.0)
    g3 = jnp.maximum(jnp.max(h3, axis=-1, keepdims=True), 0.0)
    lg = hat[c:c + 1, :]
    lo = jnp.maximum(lg, _NEG_SLOPE * lg)
    t = jnp.exp(lo - jnp.max(lo))
    tht = h3 * t

    ones_row = jnp.ones((1, n), jnp.float32)
    rt = jnp.concatenate([tht, t, ones_row], axis=0)
    psT = lax.dot_general(
        rt, adj, dimension_numbers=(((1,), (1,)), ((), ())),
        preferred_element_type=jnp.float32,
    )
    invr = 1.0 / psT[c:c + 1, :]
    rdeg = 1.0 / jnp.maximum(psT[c + 1:c + 2, :], 1.0)
    sT = psT[:c, :] * invr
    mT = sT * rdeg
    ms6 = jnp.concatenate([mT, sT], axis=0).T

    id2 = jnp.concatenate([invr, psT[c + 1:c + 2, :]], axis=0).T
    inv_col = id2[:, 0:1]
    deg_col = id2[:, 1:2]

    ub = (tht - f3).astype(jnp.bfloat16)
    vb = (tht - g3).astype(jnp.bfloat16)
    blk = 64
    parts = []
    for r0 in range(0, n, blk):
        ab = adj[r0:r0 + blk, :].astype(jnp.bfloat16)
        mms = []
        for ch in range(c):
            mms.append(jnp.max(ab * ub[ch:ch + 1, :], axis=-1, keepdims=True))
        for ch in range(c):
            mms.append(jnp.min(ab * vb[ch:ch + 1, :], axis=-1, keepdims=True))
        parts.append(jnp.concatenate(mms, axis=-1))
    fg6 = jnp.concatenate([f3, g3], axis=0).T
    mm6 = (jnp.concatenate(parts, axis=0).astype(jnp.float32) + fg6) * inv_col

    agg = jnp.concatenate(
        [ms6[:, :c], mm6[:, :c], ms6[:, c:], mm6[:, c:]], axis=-1)
    agg = jnp.where(deg_col > 0.0, agg, jnp.zeros_like(agg))

    rows = lax.broadcasted_iota(jnp.int32, (c, 4 * c), 0)
    cols = lax.broadcasted_iota(jnp.int32, (c, 4 * c), 1)
    skip_sel = (cols % c == rows).astype(jnp.float32)
    skip = jnp.dot(x, skip_sel, preferred_element_type=jnp.float32)
    out_ref[...] = agg + skip


@jax.jit
def _forward(x, adj, w_aug, b_aug):
    bsz, n, c_in = x.shape
    c = w_aug.shape[1] - 1
    return pl.pallas_call(
        _gconv_kernel,
        out_shape=jax.ShapeDtypeStruct((bsz, n, 4 * c), jnp.float32),
        grid=(bsz,),
        in_specs=[
            pl.BlockSpec((None, n, c_in), lambda i: (i, 0, 0)),
            pl.BlockSpec((None, n, n), lambda i: (i, 0, 0)),
            pl.BlockSpec((c_in, c + 1), lambda i: (0, 0)),
            pl.BlockSpec((1, c + 1), lambda i: (0, 0)),
        ],
        out_specs=pl.BlockSpec((None, n, 4 * c), lambda i: (i, 0, 0)),
        compiler_params=pltpu.CompilerParams(dimension_semantics=("parallel",)),
    )(x, adj, w_aug, b_aug)


def kernel(x, adj, w_aug, b_aug):
    return _forward(x, adj, w_aug, b_aug)
